# baseline (speedup 1.0000x reference)
import functools

import jax
import jax.numpy as jnp
from jax import lax
from jax.experimental import pallas as pl
from jax.experimental.pallas import tpu as pltpu
from jax.experimental.pallas import tpu_sc as plsc

_N_NODES = 10000
_N_EDGES = 320000
_D_EDGE = 16
_NC = 2
_NS = 16
_L = 16
_NW = _NC * _NS
_EPW = _N_EDGES // _NW
_NPAD = 10240
_CHUNK_E = 2000
_NCHUNK = _EPW // _CHUNK_E
_CFLAT = _CHUNK_E * _D_EDGE


def _sc_partials(nodes, edge_flat, gi_flat):
    mesh = plsc.VectorSubcoreMesh(core_axis_name="c", subcore_axis_name="s")

    @functools.partial(
        pl.kernel,
        out_type=jax.ShapeDtypeStruct((_NW, _NPAD), jnp.float32),
        mesh=mesh,
        compiler_params=pltpu.CompilerParams(needs_layout_passes=False),
        scratch_types=[
            pltpu.VMEM((_N_NODES,), jnp.float32),
            pltpu.VMEM((2 * _EPW,), jnp.int32),
            pltpu.VMEM((_CFLAT,), jnp.float32),
            pltpu.VMEM((_CFLAT,), jnp.float32),
            pltpu.VMEM((_NPAD,), jnp.float32),
            pltpu.SemaphoreType.DMA,
            pltpu.SemaphoreType.DMA,
        ],
    )
    def k(nodes_hbm, ef_hbm, gi_hbm, out_hbm,
          nodes_v, gi_v, ec0, ec1, acc_v, sem0, sem1):
        c = lax.axis_index("c")
        s = lax.axis_index("s")
        wid = s * _NC + c
        base = wid * _EPW
        flat_base = base * _D_EDGE

        bufs = (ec0, ec1)
        sems = (sem0, sem1)
        cps = [None, None]
        cps[0] = pltpu.async_copy(
            ef_hbm.at[pl.ds(flat_base, _CFLAT)], bufs[0], sems[0])

        pltpu.sync_copy(nodes_hbm, nodes_v)
        pltpu.sync_copy(gi_hbm.at[pl.ds(2 * base, 2 * _EPW)], gi_v)

        zeros = jnp.zeros((_L,), jnp.float32)

        def zero_body(i, carry):
            acc_v[pl.ds(i * _L, _L)] = zeros
            return carry

        lax.fori_loop(0, _NPAD // _L, zero_body, 0)

        iota = lax.iota(jnp.int32, _L)
        iota2 = iota * 2
        iota16 = iota * _D_EDGE

        for ci in range(_NCHUNK):
            if ci + 1 < _NCHUNK:
                nb = (ci + 1) % 2
                cps[nb] = pltpu.async_copy(
                    ef_hbm.at[pl.ds(flat_base + (ci + 1) * _CFLAT, _CFLAT)],
                    bufs[nb], sems[nb])
            cps[ci % 2].wait()
            buf = bufs[ci % 2]
            gi_off = 2 * ci * _CHUNK_E

            def body(j, carry, buf=buf, gi_off=gi_off):
                flat = gi_off + 2 * (j * _L) + iota2
                s_idx = plsc.load_gather(gi_v, [flat])
                r_idx = plsc.load_gather(gi_v, [flat + 1])
                e = plsc.load_gather(buf, [j * (_L * _D_EDGE) + iota16])
                ns = plsc.load_gather(nodes_v, [s_idx])
                nr = plsc.load_gather(nodes_v, [r_idx])
                upd = (ns - nr) / e
                plsc.addupdate_scatter(acc_v, [r_idx], upd)
                return carry

            lax.fori_loop(0, _CHUNK_E // _L, body, 0)

        pltpu.sync_copy(acc_v, out_hbm.at[wid])

    return k(nodes, edge_flat, gi_flat)


def _tc_combine(partials):
    def body(p_ref, o_ref):
        o_ref[...] = jnp.sum(p_ref[...], axis=0, keepdims=True)

    return pl.pallas_call(
        body,
        out_shape=jax.ShapeDtypeStruct((1, _NPAD), jnp.float32),
    )(partials)


def kernel(input_node, input_edge, graph_index):
    nodes = input_node[:, 0]
    partials = _sc_partials(
        nodes, input_edge.reshape(-1), graph_index.reshape(-1))
    summed = _tc_combine(partials)
    return summed.reshape(-1)[:_N_NODES]
